# baseline (speedup 1.0000x reference)
import functools

import jax
import jax.numpy as jnp
from jax import lax
from jax.experimental import pallas as pl
from jax.experimental.pallas import tpu as pltpu
from jax.experimental.pallas import tpu_sc as plsc


def _capacity(num_tokens: int, e: int) -> tuple[int, int]:
    sqrt_tokens = int(float(num_tokens) ** 0.5)
    target_group_size = min(4096, max(32, sqrt_tokens))
    num_groups = (num_tokens + target_group_size - 1) // target_group_size
    group_size = target_group_size
    total_tokens = num_groups * group_size
    tokens_per_expert = total_tokens / max(1, e)
    capacity_from_factor = int(2.0 * tokens_per_expert)
    min_capacity = max(max(8, group_size), int(total_tokens * 0.001))
    max_capacity = min(group_size * 32, int(total_tokens * 0.1))
    expert_capacity = min(max_capacity, max(capacity_from_factor, min_capacity))
    return total_tokens, expert_capacity


def _router_kernel(k, x_ref, gw_ref, gb_ref, temp_ref, scores_ref):
    e = gw_ref.shape[1]
    n = x_ref.shape[0]
    logits = jnp.dot(x_ref[...], gw_ref[...],
                     preferred_element_type=jnp.float32) + gb_ref[...]
    safe_temp = jnp.maximum(temp_ref[0, 0], 0.1)
    logits = logits / safe_temp
    logits = logits - jnp.max(logits, axis=1, keepdims=True)
    unnorm = jnp.exp(logits)
    probs = unnorm / jnp.sum(unnorm, axis=1, keepdims=True)

    bits = lax.bitcast_convert_type(probs, jnp.int32)
    kk = jnp.int32(k)

    def count_ge(t):
        return jnp.sum((bits >= t).astype(jnp.int32), axis=0, keepdims=True)

    def body1(_, carry):
        lo, hi = carry
        mid = (lo + hi) >> 1
        ge = count_ge(mid) >= kk
        return (jnp.where(ge, mid, lo), jnp.where(ge, hi, mid))

    lo0 = jnp.zeros((1, e), jnp.int32)
    hi0 = jnp.full((1, e), jnp.int32(0x3F800001))
    tbits, _ = lax.fori_loop(0, 31, body1, (lo0, hi0))

    r = kk - count_ge(tbits + 1)
    tie = bits == tbits
    idxv = lax.broadcasted_iota(jnp.int32, (n, e), 0)

    def body2(_, carry):
        lo, hi = carry
        mid = (lo + hi) >> 1
        cnt = jnp.sum((tie & (idxv < mid)).astype(jnp.int32), axis=0,
                      keepdims=True)
        ge = cnt >= r
        return (jnp.where(ge, lo, mid), jnp.where(ge, mid, hi))

    lo0 = jnp.zeros((1, e), jnp.int32)
    hi0 = jnp.full((1, e), jnp.int32(n))
    _, m = lax.fori_loop(0, 14, body2, (lo0, hi0))

    mask = (bits > tbits) | (tie & (idxv < m))
    scores_ne = jnp.where(mask, probs, 0.0)
    eye = (lax.broadcasted_iota(jnp.int32, (e, e), 0) ==
           lax.broadcasted_iota(jnp.int32, (e, e), 1)).astype(jnp.float32)
    scores_ref[...] = lax.dot_general(eye, scores_ne,
                                      (((1,), (1,)), ((), ())),
                                      preferred_element_type=jnp.float32)


def _compact_gather_body(n, cap_pad, e, rows_per_tile, chunks,
                         scores_hbm, x_hbm, idx_hbm, cs_hbm, xe_hbm,
                         row_v, idx_v, cs_v, b0, b1, b2, b3, s0, s1, s2, s3):
    cid = lax.axis_index("c")
    sid = lax.axis_index("s")

    @pl.when(sid < e)
    def _():
        pltpu.sync_copy(scores_hbm.at[sid], row_v)

        def zero_body(i, _):
            idx_v[pl.ds(i * 16, 16)] = jnp.zeros((16,), jnp.int32)
            cs_v[pl.ds(i * 16, 16)] = jnp.zeros((16,), jnp.float32)
            return 0

        lax.fori_loop(0, cap_pad // 16, zero_body, 0)

        def body(i, cnt):
            v = row_v[pl.ds(i * 16, 16)]
            msk = v > 0.0
            mi = msk.astype(jnp.int32)
            pos = cnt + plsc.cumsum(mi) - mi
            toks = lax.iota(jnp.int32, 16) + i * 16
            plsc.store_scatter(idx_v, [pos], toks, mask=msk)
            plsc.store_scatter(cs_v, [pos], v, mask=msk)
            return cnt + jnp.sum(mi)

        lax.fori_loop(0, n // 16, body, jnp.int32(0))
        ne = e * cap_pad
        pltpu.sync_copy(idx_v, idx_hbm.at[pl.ds(cid * ne + sid * cap_pad,
                                                cap_pad)])

        @pl.when(cid == 0)
        def _():
            pltpu.sync_copy(cs_v, cs_hbm.at[sid])

    plsc.subcore_barrier()

    ne = e * cap_pad
    grow = (cid * 16 + sid) * rows_per_tile
    pltpu.sync_copy(idx_hbm.at[pl.ds(cid * ne + grow, rows_per_tile)],
                    idx_v.at[pl.ds(0, rows_per_tile)])
    bufs = (b0, b1, b2, b3)
    sems = (s0, s1, s2, s3)
    gd = [pltpu.async_copy(x_hbm.at[idx_v.at[pl.ds(o, c)]],
                           bufs[j].at[pl.ds(0, c)], sems[j])
          for j, (o, c) in enumerate(chunks)]
    wd = []
    for j, (o, c) in enumerate(chunks):
        gd[j].wait()
        wd.append(pltpu.async_copy(bufs[j].at[pl.ds(0, c)],
                                   xe_hbm.at[pl.ds(grow + o, c)], sems[j]))
    for w in wd:
        w.wait()


def _expert_ff_kernel(xe_ref, wk_ref, bk_ref, wv_ref, bv_ref, s_ref, o_ref):
    xb = xe_ref[...].astype(jnp.bfloat16)
    hb = jnp.dot(xb, wk_ref[0].astype(jnp.bfloat16),
                 preferred_element_type=jnp.float32)
    hb = jax.nn.gelu(hb + bk_ref[0]).astype(jnp.bfloat16)
    y = jnp.dot(hb, wv_ref[0].astype(jnp.bfloat16),
                preferred_element_type=jnp.float32) + bv_ref[0]
    o_ref[...] = (y * s_ref[0]).astype(jnp.bfloat16)


def _shared_scatter_kernel(tb, x_ref, wk_ref, bk_ref, wv_ref, bv_ref,
                           idx_ref, ye_ref, o_ref):
    xb = x_ref[...].astype(jnp.bfloat16)
    h = jnp.dot(xb, wk_ref[...].astype(jnp.bfloat16),
                preferred_element_type=jnp.float32)
    h = jax.nn.gelu(h + bk_ref[...]).astype(jnp.bfloat16)
    sh = jnp.dot(h, wv_ref[...].astype(jnp.bfloat16),
                 preferred_element_type=jnp.float32) + bv_ref[...]
    t = pl.program_id(0)
    tvec = t * tb + lax.broadcasted_iota(jnp.int32, (tb, 1), 0)
    oh = (idx_ref[...] == tvec).astype(jnp.bfloat16)
    o_ref[...] = sh + jnp.dot(oh, ye_ref[...],
                              preferred_element_type=jnp.float32)


def kernel(x, gate_W, gate_b, temperature, Wk, bk, Wv, bv, sWk, sbk, sWv, sbv):
    b, s, d = x.shape
    e = gate_W.shape[1]
    h = Wk.shape[2]
    num_tokens = b * s
    total_tokens, cap = _capacity(num_tokens, e)
    x_flat = x.reshape(num_tokens, d)
    if total_tokens != num_tokens:
        x_flat = jnp.pad(x_flat, ((0, total_tokens - num_tokens), (0, 0)))
    n = total_tokens
    cap_pad = ((cap + 103) // 104) * 104
    ne = e * cap_pad

    scores = pl.pallas_call(
        functools.partial(_router_kernel, cap),
        out_shape=jax.ShapeDtypeStruct((e, n), jnp.float32),
    )(x_flat, gate_W, gate_b.reshape(1, e), temperature.reshape(1, 1))

    mesh = plsc.VectorSubcoreMesh(core_axis_name="c", subcore_axis_name="s")
    sc_params = pltpu.CompilerParams(needs_layout_passes=False)

    rows_per_tile = ne // 32
    cbase = (rows_per_tile // 4) // 8 * 8
    chunks = [(j * cbase, cbase) for j in range(3)]
    chunks.append((3 * cbase, rows_per_tile - 3 * cbase))
    idx2, cscores, xe = pl.kernel(
        functools.partial(_compact_gather_body, n, cap_pad, e, rows_per_tile,
                          tuple(chunks)),
        out_type=[jax.ShapeDtypeStruct((2 * ne,), jnp.int32),
                  jax.ShapeDtypeStruct((e, cap_pad), jnp.float32),
                  jax.ShapeDtypeStruct((ne, d), jnp.float32)],
        mesh=mesh,
        scratch_types=[pltpu.VMEM((n,), jnp.float32),
                       pltpu.VMEM((cap_pad,), jnp.int32),
                       pltpu.VMEM((cap_pad,), jnp.float32)]
        + [pltpu.VMEM((c, d), jnp.float32) for _, c in chunks]
        + [pltpu.SemaphoreType.DMA] * 4,
        compiler_params=sc_params,
    )(scores, x_flat)

    ye = pl.pallas_call(
        _expert_ff_kernel,
        grid=(e,),
        in_specs=[
            pl.BlockSpec((cap_pad, d), lambda ei: (ei, 0)),
            pl.BlockSpec((1, d, h), lambda ei: (ei, 0, 0)),
            pl.BlockSpec((1, 1, h), lambda ei: (ei, 0, 0)),
            pl.BlockSpec((1, h, d), lambda ei: (ei, 0, 0)),
            pl.BlockSpec((1, 1, d), lambda ei: (ei, 0, 0)),
            pl.BlockSpec((1, cap_pad, 1), lambda ei: (ei, 0, 0)),
        ],
        out_specs=pl.BlockSpec((cap_pad, d), lambda ei: (ei, 0)),
        out_shape=jax.ShapeDtypeStruct((ne, d), jnp.bfloat16),
    )(xe, Wk, bk.reshape(e, 1, h), Wv, bv.reshape(e, 1, d),
      cscores.reshape(e, cap_pad, 1))

    tb = 512
    ntb = n // tb
    out = pl.pallas_call(
        functools.partial(_shared_scatter_kernel, tb),
        grid=(ntb,),
        in_specs=[
            pl.BlockSpec((tb, d), lambda t: (t, 0)),
            pl.BlockSpec((d, h), lambda t: (0, 0)),
            pl.BlockSpec((1, h), lambda t: (0, 0)),
            pl.BlockSpec((h, d), lambda t: (0, 0)),
            pl.BlockSpec((1, d), lambda t: (0, 0)),
            pl.BlockSpec((1, ne), lambda t: (0, 0)),
            pl.BlockSpec((ne, d), lambda t: (0, 0)),
        ],
        out_specs=pl.BlockSpec((tb, d), lambda t: (t, 0)),
        out_shape=jax.ShapeDtypeStruct((n, d), jnp.float32),
    )(x_flat, sWk, sbk.reshape(1, h), sWv, sbv.reshape(1, d),
      idx2[:ne].reshape(1, ne), ye)

    return out[:num_tokens].reshape(b, s, d)


# speedup vs baseline: 1.0929x; 1.0929x over previous
import functools

import jax
import jax.numpy as jnp
from jax import lax
from jax.experimental import pallas as pl
from jax.experimental.pallas import tpu as pltpu
from jax.experimental.pallas import tpu_sc as plsc


def _capacity(num_tokens: int, e: int) -> tuple[int, int]:
    sqrt_tokens = int(float(num_tokens) ** 0.5)
    target_group_size = min(4096, max(32, sqrt_tokens))
    num_groups = (num_tokens + target_group_size - 1) // target_group_size
    group_size = target_group_size
    total_tokens = num_groups * group_size
    tokens_per_expert = total_tokens / max(1, e)
    capacity_from_factor = int(2.0 * tokens_per_expert)
    min_capacity = max(max(8, group_size), int(total_tokens * 0.001))
    max_capacity = min(group_size * 32, int(total_tokens * 0.1))
    expert_capacity = min(max_capacity, max(capacity_from_factor, min_capacity))
    return total_tokens, expert_capacity


def _router_kernel(k, x_ref, gw_ref, gb_ref, temp_ref, scores_ref):
    e = gw_ref.shape[1]
    n = x_ref.shape[0]
    logits = jnp.dot(x_ref[...], gw_ref[...],
                     preferred_element_type=jnp.float32) + gb_ref[...]
    safe_temp = jnp.maximum(temp_ref[0, 0], 0.1)
    logits = logits / safe_temp
    logits = logits - jnp.max(logits, axis=1, keepdims=True)
    unnorm = jnp.exp(logits)
    probs_ne = unnorm / jnp.sum(unnorm, axis=1, keepdims=True)
    eye = (lax.broadcasted_iota(jnp.int32, (e, e), 0) ==
           lax.broadcasted_iota(jnp.int32, (e, e), 1)).astype(jnp.float32)
    probs = lax.dot_general(eye, probs_ne, (((1,), (1,)), ((), ())),
                            preferred_element_type=jnp.float32)

    bits = lax.bitcast_convert_type(probs, jnp.int32)
    kk = jnp.int32(k)

    def count_ge(t):
        return jnp.sum((bits >= t).astype(jnp.int32), axis=1, keepdims=True)

    def body1(_, carry):
        lo, hi = carry
        mid = (lo + hi) >> 1
        ge = count_ge(mid) >= kk
        return (jnp.where(ge, mid, lo), jnp.where(ge, hi, mid))

    lo0 = jnp.zeros((e, 1), jnp.int32)
    hi0 = jnp.full((e, 1), jnp.int32(0x3F800001))
    tbits, _ = lax.fori_loop(0, 31, body1, (lo0, hi0))

    r = kk - count_ge(tbits + 1)
    tie = bits == tbits
    idxv = lax.broadcasted_iota(jnp.int32, (e, n), 1)

    def body2(_, carry):
        lo, hi = carry
        mid = (lo + hi) >> 1
        cnt = jnp.sum((tie & (idxv < mid)).astype(jnp.int32), axis=1,
                      keepdims=True)
        ge = cnt >= r
        return (jnp.where(ge, lo, mid), jnp.where(ge, mid, hi))

    lo0 = jnp.zeros((e, 1), jnp.int32)
    hi0 = jnp.full((e, 1), jnp.int32(n))
    _, m = lax.fori_loop(0, 14, body2, (lo0, hi0))

    mask = (bits > tbits) | (tie & (idxv < m))
    scores_ref[...] = jnp.where(mask, probs, 0.0)


def _compact_gather_body(n, cap_pad, e, rows_per_tile, chunks,
                         scores_hbm, x_hbm, idx_hbm, cs_hbm, xe_hbm,
                         row_v, idx_v, cs_v, b0, b1, b2, b3, s0, s1, s2, s3):
    cid = lax.axis_index("c")
    sid = lax.axis_index("s")

    @pl.when(sid < e)
    def _():
        pltpu.sync_copy(scores_hbm.at[sid], row_v)

        def zero_body(i, _):
            idx_v[pl.ds(i * 16, 16)] = jnp.zeros((16,), jnp.int32)
            cs_v[pl.ds(i * 16, 16)] = jnp.zeros((16,), jnp.float32)
            return 0

        lax.fori_loop(0, cap_pad // 16, zero_body, 0)

        def body(i, cnt):
            v = row_v[pl.ds(i * 16, 16)]
            msk = v > 0.0
            mi = msk.astype(jnp.int32)
            pos = cnt + plsc.cumsum(mi) - mi
            toks = lax.iota(jnp.int32, 16) + i * 16
            plsc.store_scatter(idx_v, [pos], toks, mask=msk)
            plsc.store_scatter(cs_v, [pos], v, mask=msk)
            return cnt + jnp.sum(mi)

        lax.fori_loop(0, n // 16, body, jnp.int32(0))
        ne = e * cap_pad
        pltpu.sync_copy(idx_v, idx_hbm.at[pl.ds(cid * ne + sid * cap_pad,
                                                cap_pad)])

        @pl.when(cid == 0)
        def _():
            pltpu.sync_copy(cs_v, cs_hbm.at[sid])

    plsc.subcore_barrier()

    ne = e * cap_pad
    grow = (cid * 16 + sid) * rows_per_tile
    pltpu.sync_copy(idx_hbm.at[pl.ds(cid * ne + grow, rows_per_tile)],
                    idx_v.at[pl.ds(0, rows_per_tile)])
    bufs = (b0, b1, b2, b3)
    sems = (s0, s1, s2, s3)
    gd = [pltpu.async_copy(x_hbm.at[idx_v.at[pl.ds(o, c)]],
                           bufs[j].at[pl.ds(0, c)], sems[j])
          for j, (o, c) in enumerate(chunks)]
    wd = []
    for j, (o, c) in enumerate(chunks):
        gd[j].wait()
        wd.append(pltpu.async_copy(bufs[j].at[pl.ds(0, c)],
                                   xe_hbm.at[pl.ds(grow + o, c)], sems[j]))
    for w in wd:
        w.wait()


def _expert_ff_kernel(xe_ref, wk_ref, bk_ref, wv_ref, bv_ref, s_ref, o_ref):
    xb = xe_ref[...].astype(jnp.bfloat16)
    hb = jnp.dot(xb, wk_ref[0].astype(jnp.bfloat16),
                 preferred_element_type=jnp.float32)
    hb = jax.nn.gelu(hb + bk_ref[0]).astype(jnp.bfloat16)
    y = jnp.dot(hb, wv_ref[0].astype(jnp.bfloat16),
                preferred_element_type=jnp.float32) + bv_ref[0]
    o_ref[...] = (y * s_ref[0]).astype(jnp.bfloat16)


def _shared_scatter_kernel(tb, x_ref, wk_ref, bk_ref, wv_ref, bv_ref,
                           idx_ref, ye_ref, o_ref):
    xb = x_ref[...].astype(jnp.bfloat16)
    h = jnp.dot(xb, wk_ref[...].astype(jnp.bfloat16),
                preferred_element_type=jnp.float32)
    h = jax.nn.gelu(h + bk_ref[...]).astype(jnp.bfloat16)
    sh = jnp.dot(h, wv_ref[...].astype(jnp.bfloat16),
                 preferred_element_type=jnp.float32) + bv_ref[...]
    t = pl.program_id(0)
    tvec = t * tb + lax.broadcasted_iota(jnp.int32, (tb, 1), 0)
    oh = (idx_ref[...] == tvec).astype(jnp.bfloat16)
    o_ref[...] = sh + jnp.dot(oh, ye_ref[...],
                              preferred_element_type=jnp.float32)


def kernel(x, gate_W, gate_b, temperature, Wk, bk, Wv, bv, sWk, sbk, sWv, sbv):
    b, s, d = x.shape
    e = gate_W.shape[1]
    h = Wk.shape[2]
    num_tokens = b * s
    total_tokens, cap = _capacity(num_tokens, e)
    x_flat = x.reshape(num_tokens, d)
    if total_tokens != num_tokens:
        x_flat = jnp.pad(x_flat, ((0, total_tokens - num_tokens), (0, 0)))
    n = total_tokens
    cap_pad = ((cap + 103) // 104) * 104
    ne = e * cap_pad

    scores = pl.pallas_call(
        functools.partial(_router_kernel, cap),
        out_shape=jax.ShapeDtypeStruct((e, n), jnp.float32),
    )(x_flat, gate_W, gate_b.reshape(1, e), temperature.reshape(1, 1))

    mesh = plsc.VectorSubcoreMesh(core_axis_name="c", subcore_axis_name="s")
    sc_params = pltpu.CompilerParams(needs_layout_passes=False)

    rows_per_tile = ne // 32
    cbase = (rows_per_tile // 4) // 8 * 8
    chunks = [(j * cbase, cbase) for j in range(3)]
    chunks.append((3 * cbase, rows_per_tile - 3 * cbase))
    idx2, cscores, xe = pl.kernel(
        functools.partial(_compact_gather_body, n, cap_pad, e, rows_per_tile,
                          tuple(chunks)),
        out_type=[jax.ShapeDtypeStruct((2 * ne,), jnp.int32),
                  jax.ShapeDtypeStruct((e, cap_pad), jnp.float32),
                  jax.ShapeDtypeStruct((ne, d), jnp.float32)],
        mesh=mesh,
        scratch_types=[pltpu.VMEM((n,), jnp.float32),
                       pltpu.VMEM((cap_pad,), jnp.int32),
                       pltpu.VMEM((cap_pad,), jnp.float32)]
        + [pltpu.VMEM((c, d), jnp.float32) for _, c in chunks]
        + [pltpu.SemaphoreType.DMA] * 4,
        compiler_params=sc_params,
    )(scores, x_flat)

    ye = pl.pallas_call(
        _expert_ff_kernel,
        grid=(e,),
        in_specs=[
            pl.BlockSpec((cap_pad, d), lambda ei: (ei, 0)),
            pl.BlockSpec((1, d, h), lambda ei: (ei, 0, 0)),
            pl.BlockSpec((1, 1, h), lambda ei: (ei, 0, 0)),
            pl.BlockSpec((1, h, d), lambda ei: (ei, 0, 0)),
            pl.BlockSpec((1, 1, d), lambda ei: (ei, 0, 0)),
            pl.BlockSpec((1, cap_pad, 1), lambda ei: (ei, 0, 0)),
        ],
        out_specs=pl.BlockSpec((cap_pad, d), lambda ei: (ei, 0)),
        out_shape=jax.ShapeDtypeStruct((ne, d), jnp.bfloat16),
    )(xe, Wk, bk.reshape(e, 1, h), Wv, bv.reshape(e, 1, d),
      cscores.reshape(e, cap_pad, 1))

    tb = 512
    ntb = n // tb
    out = pl.pallas_call(
        functools.partial(_shared_scatter_kernel, tb),
        grid=(ntb,),
        in_specs=[
            pl.BlockSpec((tb, d), lambda t: (t, 0)),
            pl.BlockSpec((d, h), lambda t: (0, 0)),
            pl.BlockSpec((1, h), lambda t: (0, 0)),
            pl.BlockSpec((h, d), lambda t: (0, 0)),
            pl.BlockSpec((1, d), lambda t: (0, 0)),
            pl.BlockSpec((1, ne), lambda t: (0, 0)),
            pl.BlockSpec((ne, d), lambda t: (0, 0)),
        ],
        out_specs=pl.BlockSpec((tb, d), lambda t: (t, 0)),
        out_shape=jax.ShapeDtypeStruct((n, d), jnp.float32),
    )(x_flat, sWk, sbk.reshape(1, h), sWv, sbv.reshape(1, d),
      idx2[:ne].reshape(1, ne), ye)

    return out[:num_tokens].reshape(b, s, d)
